# Initial kernel scaffold; baseline (speedup 1.0000x reference)
import jax
import jax.numpy as jnp
from jax.experimental import pallas as pl


def kernel(h, edge_index, params):
    raise NotImplementedError("write your pallas kernel here")


# speedup vs baseline: 4.4707x; 4.4707x over previous
import functools

import jax
import jax.numpy as jnp
from jax import lax
from jax.experimental import pallas as pl
from jax.experimental.pallas import tpu as pltpu
from jax.experimental.pallas import tpu_sc as plsc

N = 10000
E = 320000
H = 128
NUM_LAYERS = 2

NC = 2
NS = 16
NW = NC * NS
EDGES_PER_TILE = E // NW
CH = 80
NCHUNK = EDGES_PER_TILE // CH
NPAD = 10240
ROWS_PER_TILE = NPAD // NS


def _sc_segment_sum_body(x_hbm, src_hbm, dst_hbm, zero_hbm, out_hbm,
                         src_v, dst_v, rows_v, acc_sh, sem):
  core = lax.axis_index("c")
  sub = lax.axis_index("s")
  wid = sub * NC + core

  pltpu.sync_copy(zero_hbm, acc_sh.at[pl.ds(sub * ROWS_PER_TILE, ROWS_PER_TILE)])
  plsc.subcore_barrier()

  base = wid * EDGES_PER_TILE

  def body(c, carry):
    off = base + c * CH
    pltpu.sync_copy(src_hbm.at[pl.ds(off, CH)], src_v)
    pltpu.sync_copy(dst_hbm.at[pl.ds(off, CH)], dst_v)
    pltpu.async_copy(x_hbm.at[src_v], rows_v, sem).wait()
    pltpu.sync_copy(rows_v, acc_sh.at[dst_v], add=True)
    return carry

  lax.fori_loop(0, NCHUNK, body, 0)
  plsc.subcore_barrier()

  r0 = sub * ROWS_PER_TILE
  pltpu.sync_copy(acc_sh.at[pl.ds(r0, ROWS_PER_TILE)],
                  out_hbm.at[core, pl.ds(r0, ROWS_PER_TILE)])


_sc_segment_sum = functools.partial(
    pl.kernel,
    mesh=plsc.VectorSubcoreMesh(core_axis_name="c", subcore_axis_name="s"),
    out_type=jax.ShapeDtypeStruct((NC, NPAD, H), jnp.float32),
    scratch_types=[
        pltpu.VMEM((CH,), jnp.int32),
        pltpu.VMEM((CH,), jnp.int32),
        pltpu.VMEM((CH, H), jnp.float32),
        pltpu.VMEM_SHARED((NPAD, H), jnp.float32),
        pltpu.SemaphoreType.DMA,
    ],
)(_sc_segment_sum_body)


def _tc_dense_body(x_ref, p0_ref, p1_ref, w0_ref, w1_ref, pp_ref, out_ref):
  def bn_relu(m, g_row, b_row):
    mean = jnp.mean(m, axis=0, keepdims=True)
    c = m - mean
    var = jnp.mean(c * c, axis=0, keepdims=True)
    y = c * lax.rsqrt(var + 1e-5) * pp_ref[g_row:g_row + 1, :] \
        + pp_ref[b_row:b_row + 1, :]
    return jnp.maximum(y, 0.0)

  neigh = p0_ref[...] + p1_ref[...]
  r = (1.0 + pp_ref[6:7, :]) * x_ref[...] + neigh
  m = jnp.dot(r, w0_ref[...], preferred_element_type=jnp.float32)
  m = bn_relu(m, 0, 1)
  m = jnp.dot(m, w1_ref[...], preferred_element_type=jnp.float32)
  m = bn_relu(m, 2, 3)
  out_ref[...] = bn_relu(m, 4, 5)


_tc_dense = pl.pallas_call(
    _tc_dense_body,
    out_shape=jax.ShapeDtypeStruct((N, H), jnp.float32),
)


def kernel(h, edge_index, params):
  src = edge_index[0]
  dst = edge_index[1]
  zero = jnp.zeros((ROWS_PER_TILE, H), jnp.float32)
  outs = [h]
  x = h
  for i in range(NUM_LAYERS):
    partials = _sc_segment_sum(x, src, dst, zero)
    partials = partials[:, :N, :]
    pp = jnp.stack([
        params[f"g_mlp_{i}"], params[f"b_mlp_{i}"],
        params[f"g_app_{i}"], params[f"b_app_{i}"],
        params[f"g_enc_{i}"], params[f"b_enc_{i}"],
        jnp.full((H,), params[f"eps_{i}"], jnp.float32),
        jnp.zeros((H,), jnp.float32),
    ])
    x = _tc_dense(x, partials[0], partials[1],
                  params[f"W0_{i}"], params[f"W1_{i}"], pp)
    outs.append(x)
  return jnp.concatenate([t.reshape(1, N, H) for t in outs], axis=-1)
